# baseline (speedup 1.0000x reference)
import jax
import jax.numpy as jnp
from jax import lax
from jax.experimental import pallas as pl
from jax.experimental.pallas import tpu as pltpu
from jax.experimental.pallas import tpu_sc as plsc
import functools

VOCAB = 1000000
DIM = 64
NC = 2
NS = 16
NW = NC * NS

BLK = 128


def _make_gather(n_seq: int, n_batch: int):
    nblk = n_seq * (n_batch // BLK)
    assert nblk % NW == 0
    bpw = nblk // NW
    assert bpw % 2 == 0
    bt_per_l = n_batch // BLK

    mesh = plsc.VectorSubcoreMesh(core_axis_name="c", subcore_axis_name="s")

    @functools.partial(
        pl.kernel,
        out_type=jax.ShapeDtypeStruct((n_seq, 8, bt_per_l, 8, BLK),
                                      jnp.float32),
        mesh=mesh,
        scratch_types=[
            pltpu.VMEM((bpw, BLK), jnp.int32),
            pltpu.VMEM((2, BLK, DIM), jnp.float32),
            pltpu.VMEM((2, 8, 1, 8, BLK), jnp.float32),
            [pltpu.SemaphoreType.DMA] * 2,
            [pltpu.SemaphoreType.DMA] * 2,
        ],
        compiler_params=pltpu.CompilerParams(use_tc_tiling_on_sc=False, needs_layout_passes=False),
    )
    def gather_kernel(idx_hbm, table_hbm, out_hbm, idx_v, rows_v,
                      outb_v, gsems, osems):
        wid = lax.axis_index("s") * NC + lax.axis_index("c")
        blk0 = wid * bpw

        pltpu.sync_copy(idx_hbm.at[pl.ds(blk0, bpw)], idx_v)

        iot = lax.iota(jnp.int32, 16)

        def issue_gather(j, b):
            pltpu.async_copy(table_hbm.at[idx_v.at[j]], rows_v.at[b],
                             gsems[b])

        def out_dst(j):
            blk = blk0 + j
            l = blk // bt_per_l
            bt = blk - l * bt_per_l
            return out_hbm.at[l, :, pl.ds(bt, 1), :, :]

        issue_gather(0, 0)

        @pl.loop(0, bpw, step=2)
        def _pair(j0):
            for b in range(2):
                j = j0 + b
                pltpu.make_async_copy(
                    table_hbm.at[idx_v.at[j]], rows_v.at[b],
                    gsems[b]).wait()
                @pl.when(j + 1 < bpw)
                def _():
                    issue_gather(j + 1, 1 - b)
                @pl.when(j0 > 0)
                def _():
                    pltpu.make_async_copy(
                        outb_v.at[b], out_dst(j - 2), osems[b]).wait()
                for bg in range(8):
                    s = pl.ds(bg * 16, 16)
                    row = iot + bg * 16
                    for ft in range(8):
                        for fi in range(8):
                            col = jnp.full((16,), ft * 8 + fi, jnp.int32)
                            v = plsc.load_gather(rows_v.at[b], [row, col])
                            outb_v[b, ft, 0, fi, s] = v
                pltpu.async_copy(outb_v.at[b], out_dst(j), osems[b])

        for b in range(2):
            pltpu.make_async_copy(
                outb_v.at[b], out_dst(bpw - 2 + b), osems[b]).wait()

    return gather_kernel


def kernel(corpus, table):
    n_batch, n_seq = corpus.shape
    idx = corpus.T.reshape(n_seq * (n_batch // BLK), BLK).astype(jnp.int32)
    out5 = _make_gather(n_seq, n_batch)(idx, table)
    return out5.transpose(2, 4, 0, 1, 3).reshape(n_batch, n_seq, DIM)


# speedup vs baseline: 1.5077x; 1.5077x over previous
import jax
import jax.numpy as jnp
from jax import lax
from jax.experimental import pallas as pl
from jax.experimental.pallas import tpu as pltpu
from jax.experimental.pallas import tpu_sc as plsc
import functools

VOCAB = 1000000
DIM = 64
NC = 2
NS = 16
NW = NC * NS

BLK = 128


def _make_gather(n_seq: int, n_batch: int):
    nblk = n_seq * (n_batch // BLK)
    assert nblk % NW == 0
    bpw = nblk // NW
    assert bpw % 2 == 0
    bt_per_l = n_batch // BLK

    mesh = plsc.VectorSubcoreMesh(core_axis_name="c", subcore_axis_name="s")

    @functools.partial(
        pl.kernel,
        out_type=jax.ShapeDtypeStruct((n_seq, 8, bt_per_l, 8, BLK),
                                      jnp.float32),
        mesh=mesh,
        scratch_types=[
            pltpu.VMEM((bpw, BLK), jnp.int32),
            pltpu.VMEM((2, BLK, DIM), jnp.float32),
            pltpu.VMEM((2, 8, 1, 8, BLK), jnp.float32),
            [pltpu.SemaphoreType.DMA] * 2,
            [pltpu.SemaphoreType.DMA] * 2,
        ],
        compiler_params=pltpu.CompilerParams(use_tc_tiling_on_sc=False, needs_layout_passes=False),
    )
    def gather_kernel(idx_hbm, table_hbm, out_hbm, idx_v, rows_v,
                      outb_v, gsems, osems):
        wid = lax.axis_index("s") * NC + lax.axis_index("c")
        blk0 = wid * bpw

        pltpu.sync_copy(idx_hbm.at[pl.ds(blk0, bpw)], idx_v)

        iot = lax.iota(jnp.int32, 16)

        def issue_gather(j, b):
            pltpu.async_copy(table_hbm.at[idx_v.at[j]], rows_v.at[b],
                             gsems[b])

        def out_dst(j):
            blk = blk0 + j
            l = blk // bt_per_l
            bt = blk - l * bt_per_l
            return out_hbm.at[l, :, pl.ds(bt, 1), :, :]

        issue_gather(0, 0)

        @pl.loop(0, bpw, step=2)
        def _pair(j0):
            for b in range(2):
                j = j0 + b
                pltpu.make_async_copy(
                    table_hbm.at[idx_v.at[j]], rows_v.at[b],
                    gsems[b]).wait()
                @pl.when(j + 1 < bpw)
                def _():
                    issue_gather(j + 1, 1 - b)
                @pl.when(j0 > 0)
                def _():
                    pltpu.make_async_copy(
                        outb_v.at[b], out_dst(j - 2), osems[b]).wait()
                for bg in range(8):
                    s = pl.ds(bg * 16, 16)
                    row = iot + bg * 16
                    for ft in range(8):
                        vs = [
                            plsc.load_gather(
                                rows_v.at[b],
                                [row, jnp.full((16,), ft * 8 + fi,
                                               jnp.int32)])
                            for fi in range(8)
                        ]
                        for fi in range(8):
                            outb_v[b, ft, 0, fi, s] = vs[fi]
                pltpu.async_copy(outb_v.at[b], out_dst(j), osems[b])

        for b in range(2):
            pltpu.make_async_copy(
                outb_v.at[b], out_dst(bpw - 2 + b), osems[b]).wait()

    return gather_kernel


def kernel(corpus, table):
    n_batch, n_seq = corpus.shape
    idx = corpus.T.reshape(n_seq * (n_batch // BLK), BLK).astype(jnp.int32)
    out5 = _make_gather(n_seq, n_batch)(idx, table)
    return out5.transpose(2, 4, 0, 1, 3).reshape(n_batch, n_seq, DIM)


# speedup vs baseline: 1.9782x; 1.3120x over previous
import jax
import jax.numpy as jnp
from jax import lax
from jax.experimental import pallas as pl
from jax.experimental.pallas import tpu as pltpu
from jax.experimental.pallas import tpu_sc as plsc
import functools

VOCAB = 1000000
DIM = 64
NC = 2
NS = 16
NW = NC * NS

BLK = 128


def _make_gather(n_seq: int, n_batch: int):
    nblk = n_seq * (n_batch // BLK)
    assert nblk % NW == 0
    bpw = nblk // NW
    assert bpw % 2 == 0
    bt_per_l = n_batch // BLK

    mesh = plsc.VectorSubcoreMesh(core_axis_name="c", subcore_axis_name="s")

    @functools.partial(
        pl.kernel,
        out_type=jax.ShapeDtypeStruct((n_seq, 8, bt_per_l, 8, BLK),
                                      jnp.float32),
        mesh=mesh,
        scratch_types=[
            pltpu.VMEM((bpw, BLK), jnp.int32),
            pltpu.VMEM((2, BLK, DIM), jnp.float32),
            pltpu.VMEM((2, 8, 1, 8, BLK), jnp.float32),
            [pltpu.SemaphoreType.DMA] * 2,
            [pltpu.SemaphoreType.DMA] * 2,
        ],
        compiler_params=pltpu.CompilerParams(use_tc_tiling_on_sc=False,
                                             needs_layout_passes=False),
    )
    def gather_kernel(idx_hbm, table_hbm, out_hbm, idx_v, rows_v,
                      outb_v, gsems, osems):
        wid = lax.axis_index("s") * NC + lax.axis_index("c")
        blk0 = wid * bpw

        pltpu.sync_copy(idx_hbm.at[pl.ds(blk0, bpw)], idx_v)

        iot = lax.iota(jnp.int32, 16)
        ones = iot < 16

        def issue_gather(j, b):
            pltpu.async_copy(table_hbm.at[idx_v.at[j]], rows_v.at[b],
                             gsems[b])

        def out_dst(j):
            blk = blk0 + j
            l = blk // bt_per_l
            bt = blk - l * bt_per_l
            return out_hbm.at[l, :, pl.ds(bt, 1), :, :]

        issue_gather(0, 0)

        @pl.loop(0, bpw, step=2)
        def _pair(j0):
            for b in range(2):
                j = j0 + b
                pltpu.make_async_copy(
                    table_hbm.at[idx_v.at[j]], rows_v.at[b],
                    gsems[b]).wait()
                @pl.when(j + 1 < bpw)
                def _():
                    issue_gather(j + 1, 1 - b)
                @pl.when(j0 > 0)
                def _():
                    pltpu.make_async_copy(
                        outb_v.at[b], out_dst(j - 2), osems[b]).wait()
                rowvs = [iot + bg * 16 for bg in range(8)]
                zero = jnp.zeros((16,), jnp.int32)

                @pl.loop(0, DIM, step=8)
                def _diag(d0):
                    for dd in range(8):
                        colv = (d0 + dd + iot) & (DIM - 1)
                        ftv = lax.shift_right_logical(colv, 3)
                        fiv = colv & 7
                        for bg in range(8):
                            v = plsc.load_gather(
                                rows_v.at[b], [rowvs[bg], colv])
                            plsc.store_scatter(
                                outb_v.at[b], [ftv, zero, fiv, rowvs[bg]],
                                v)
                pltpu.async_copy(outb_v.at[b], out_dst(j), osems[b])

        for b in range(2):
            pltpu.make_async_copy(
                outb_v.at[b], out_dst(bpw - 2 + b), osems[b]).wait()

    return gather_kernel


def kernel(corpus, table):
    n_batch, n_seq = corpus.shape
    idx = corpus.T.reshape(n_seq * (n_batch // BLK), BLK).astype(jnp.int32)
    out5 = _make_gather(n_seq, n_batch)(idx, table)
    return out5.transpose(2, 4, 0, 1, 3).reshape(n_batch, n_seq, DIM)
